# Initial kernel scaffold; baseline (speedup 1.0000x reference)
import jax
import jax.numpy as jnp
from jax.experimental import pallas as pl


def kernel(feature, edge_index):
    raise NotImplementedError("write your pallas kernel here")


# speedup vs baseline: 5.6453x; 5.6453x over previous
import jax
import jax.numpy as jnp
from jax import lax
from jax.experimental import pallas as pl
from jax.experimental.pallas import tpu as pltpu
from jax.experimental.pallas import tpu_sc as plsc

N = 10000
FD = 256
HALF = 128
E = 160000
KHOP = 8
NS = 16
NC = 2
L = 16
CH = 80
NCH = N // CH
MAXC = 8
ZR = 16
EPT = E // NS
EC = 80
NEC = EPT // EC


def _body(f_hbm, src_hbm, dst_hbm, out_hbm, y_hbm,
          acc_sp,
          dinv_b, rows_b, srcb, dstb, gidxb, zb, accb, yb, sem):
  c = lax.axis_index("c")
  s = lax.axis_index("s")
  cN = c * N
  e_base = s * EPT

  fzero = jnp.zeros((L,), jnp.float32)
  fone = jnp.ones((L,), jnp.float32)

  def fill_zb(r, carry):
    for g in range(HALF // L):
      zb[r, pl.ds(g * L, L)] = fzero
    return carry
  lax.fori_loop(0, ZR, fill_zb, 0)

  def fill_ones(r, carry):
    for g in range(HALF // L):
      rows_b[r, pl.ds(g * L, L)] = fone
    return carry
  lax.fori_loop(0, EC, fill_ones, 0)

  def zero_acc_chunk(r0):
    def zbody(z, carry):
      pltpu.sync_copy(zb, acc_sp.at[pl.ds(r0 + z * ZR, ZR), :])
      return carry
    lax.fori_loop(0, CH // ZR, zbody, 0)

  nchunks = (NCH - 1 - s) // NS + 1

  def zinit_body(i, carry):
    zero_acc_chunk((s + NS * i) * CH)
    return carry
  lax.fori_loop(0, nchunks, zinit_body, 0)

  plsc.subcore_barrier()

  def deg_body(i, carry):
    e0 = e_base + i * EC
    pltpu.sync_copy(dst_hbm.at[pl.ds(e0, EC)], dstb)
    pltpu.sync_copy(rows_b, acc_sp.at[dstb], add=True)
    return carry
  lax.fori_loop(0, NEC, deg_body, 0)

  plsc.subcore_barrier()

  def dinv_chunk(i, carry):
    cid = s + NS * i
    pltpu.sync_copy(acc_sp.at[pl.ds(cid * CH, CH), :], accb)

    def dinv_body(r, rcarry):
      v = accb[r, pl.ds(0, L)] + 1.0
      g = jnp.full((L,), 1.0, jnp.float32)
      for j in range(1, 19):
        g = jnp.where(v >= float(1 << j),
                      jnp.full((L,), 2.0 ** (-0.5 * j), jnp.float32), g)
      for _ in range(5):
        g = g * (1.5 - 0.5 * v * g * g)
      dinv_b[i * CH + r, :] = g
      return rcarry
    lax.fori_loop(0, CH, dinv_body, 0)

    zero_acc_chunk(cid * CH)
    return carry
  lax.fori_loop(0, nchunks, dinv_chunk, 0)

  def p0_chunk(i, carry):
    r0 = (s + NS * i) * CH
    pltpu.sync_copy(f_hbm.at[pl.ds(r0, CH), pl.ds(c * HALF, HALF)], accb)

    def rowf(r, rcarry):
      d = dinv_b[i * CH + r, :]
      for g in range(HALF // L):
        sl = pl.ds(g * L, L)
        yb[r, sl] = d * accb[r, sl]
      return rcarry
    lax.fori_loop(0, CH, rowf, 0)

    pltpu.sync_copy(accb, out_hbm.at[pl.ds(r0, CH), pl.ds(c * HALF, HALF)])
    pltpu.sync_copy(yb, y_hbm.at[pl.ds(cN + r0, CH), :])
    return carry
  lax.fori_loop(0, nchunks, p0_chunk, 0)

  plsc.subcore_barrier()

  for k in range(1, KHOP + 1):
    def edge_body(i, carry):
      e0 = e_base + i * EC
      pltpu.sync_copy(src_hbm.at[pl.ds(e0, EC)], srcb)
      pltpu.sync_copy(dst_hbm.at[pl.ds(e0, EC)], dstb)
      for j in range(EC // L):
        sl = pl.ds(j * L, L)
        gidxb[sl] = srcb[sl] + cN
      pltpu.async_copy(y_hbm.at[gidxb], rows_b, sem).wait()
      pltpu.sync_copy(rows_b, acc_sp.at[dstb], add=True)
      return carry
    lax.fori_loop(0, NEC, edge_body, 0)

    plsc.subcore_barrier()

    col0 = k * FD + c * HALF

    def pp_chunk(i, carry):
      r0 = (s + NS * i) * CH
      pltpu.sync_copy(acc_sp.at[pl.ds(r0, CH), :], accb)
      pltpu.sync_copy(y_hbm.at[pl.ds(cN + r0, CH), :], yb)

      def rowf(r, rcarry):
        d = dinv_b[i * CH + r, :]
        for g in range(HALF // L):
          sl = pl.ds(g * L, L)
          x = d * (accb[r, sl] + yb[r, sl])
          accb[r, sl] = x
          yb[r, sl] = d * x
        return rcarry
      lax.fori_loop(0, CH, rowf, 0)

      pltpu.sync_copy(accb, out_hbm.at[pl.ds(r0, CH), pl.ds(col0, HALF)])
      pltpu.sync_copy(yb, y_hbm.at[pl.ds(cN + r0, CH), :])
      zero_acc_chunk(r0)
      return carry
    lax.fori_loop(0, nchunks, pp_chunk, 0)

    plsc.subcore_barrier()


@jax.jit
def _lgcn(feature, src, dst):
  mesh = plsc.VectorSubcoreMesh(core_axis_name="c", subcore_axis_name="s")
  out, _ = pl.kernel(
      _body,
      out_type=(
          jax.ShapeDtypeStruct((N, (KHOP + 1) * FD), jnp.float32),
          jax.ShapeDtypeStruct((NC * N, HALF), jnp.float32),
      ),
      mesh=mesh,
      compiler_params=pltpu.CompilerParams(use_tc_tiling_on_sc=False),
      scratch_types=[
          pltpu.VMEM_SHARED((N, HALF), jnp.float32),
          pltpu.VMEM((MAXC * CH, L), jnp.float32),
          pltpu.VMEM((EC, HALF), jnp.float32),
          pltpu.VMEM((EC,), jnp.int32),
          pltpu.VMEM((EC,), jnp.int32),
          pltpu.VMEM((EC,), jnp.int32),
          pltpu.VMEM((ZR, HALF), jnp.float32),
          pltpu.VMEM((CH, HALF), jnp.float32),
          pltpu.VMEM((CH, HALF), jnp.float32),
          pltpu.SemaphoreType.DMA,
      ],
  )(feature, src, dst)
  return out


def kernel(feature, edge_index):
  ei = edge_index.astype(jnp.int32)
  return _lgcn(feature, ei[0], ei[1])


# speedup vs baseline: 8.3915x; 1.4865x over previous
import jax
import jax.numpy as jnp
from jax import lax
from jax.experimental import pallas as pl
from jax.experimental.pallas import tpu as pltpu
from jax.experimental.pallas import tpu_sc as plsc

N = 10000
FD = 256
HALF = 128
E = 160000
KHOP = 8
NS = 16
NC = 2
L = 16
CH = 80
NCH = N // CH
ZR = 16
EPT = E // NS
EC = 80
NEC = EPT // EC
NPAIR = NEC // 2


def _body(f_hbm, src_hbm, dst_hbm, out_hbm, y_hbm,
          acc_sp,
          dinv_b, rows0, rows1, src0, src1, dst0, dst1, gidx0, gidx1,
          zb, accb, sg0, sg1, ss0, ss1):
  c = lax.axis_index("c")
  s = lax.axis_index("s")
  cN = c * N
  e_base = s * EPT

  rows = (rows0, rows1)
  srcb = (src0, src1)
  dstb = (dst0, dst1)
  gidxb = (gidx0, gidx1)
  sg = (sg0, sg1)
  ss = (ss0, ss1)

  fzero = jnp.zeros((L,), jnp.float32)
  fone = jnp.ones((L,), jnp.float32)

  def fill_zb(r, carry):
    for g in range(HALF // L):
      zb[r, pl.ds(g * L, L)] = fzero
    return carry
  lax.fori_loop(0, ZR, fill_zb, 0)

  def fill_ones(r, carry):
    for g in range(HALF // L):
      rows0[r, pl.ds(g * L, L)] = fone
    return carry
  lax.fori_loop(0, EC, fill_ones, 0)

  def zero_acc_chunk(r0):
    def zbody(z, carry):
      pltpu.sync_copy(zb, acc_sp.at[pl.ds(r0 + z * ZR, ZR), :])
      return carry
    lax.fori_loop(0, CH // ZR, zbody, 0)

  nchunks = (NCH - 1 - s) // NS + 1

  def zinit_body(i, carry):
    zero_acc_chunk((s + NS * i) * CH)
    return carry
  lax.fori_loop(0, nchunks, zinit_body, 0)

  plsc.subcore_barrier()

  def deg_pair(h, carry):
    for b in range(2):
      i = 2 * h + b

      @pl.when(h > 0)
      def _():
        pltpu.make_async_copy(rows0, acc_sp.at[dstb[b]], ss[b]).wait()
      pltpu.sync_copy(dst_hbm.at[pl.ds(e_base + i * EC, EC)], dstb[b])
      pltpu.async_copy(rows0, acc_sp.at[dstb[b]], ss[b], add=True)
    return carry
  lax.fori_loop(0, NPAIR, deg_pair, 0)
  pltpu.make_async_copy(rows0, acc_sp.at[dstb[0]], ss[0]).wait()
  pltpu.sync_copy(dst_hbm.at[pl.ds(e_base + (NEC - 1) * EC, EC)], dstb[0])
  pltpu.async_copy(rows0, acc_sp.at[dstb[0]], ss[0], add=True)
  pltpu.make_async_copy(rows0, acc_sp.at[dstb[0]], ss[0]).wait()
  pltpu.make_async_copy(rows0, acc_sp.at[dstb[1]], ss[1]).wait()

  plsc.subcore_barrier()

  def dinv_chunk(i, carry):
    cid = s + NS * i
    pltpu.sync_copy(acc_sp.at[pl.ds(cid * CH, CH), :], accb)

    def dinv_body(r, rcarry):
      v = accb[r, pl.ds(0, L)] + 1.0
      g = jnp.full((L,), 1.0, jnp.float32)
      for j in range(1, 19):
        g = jnp.where(v >= float(1 << j),
                      jnp.full((L,), 2.0 ** (-0.5 * j), jnp.float32), g)
      for _ in range(5):
        g = g * (1.5 - 0.5 * v * g * g)
      dinv_b[i * CH + r, :] = g
      return rcarry
    lax.fori_loop(0, CH, dinv_body, 0)

    zero_acc_chunk(cid * CH)
    return carry
  lax.fori_loop(0, nchunks, dinv_chunk, 0)

  def p0_chunk(i, carry):
    r0 = (s + NS * i) * CH
    pltpu.sync_copy(f_hbm.at[pl.ds(r0, CH), pl.ds(c * HALF, HALF)], accb)

    def rowf(r, rcarry):
      d = dinv_b[i * CH + r, :]
      for g in range(HALF // L):
        sl = pl.ds(g * L, L)
        rows0[r, sl] = d * accb[r, sl]
      return rcarry
    lax.fori_loop(0, CH, rowf, 0)

    pltpu.sync_copy(accb, out_hbm.at[pl.ds(r0, CH), pl.ds(c * HALF, HALF)])
    pltpu.sync_copy(rows0, y_hbm.at[pl.ds(cN + r0, CH), :])
    return carry
  lax.fori_loop(0, nchunks, p0_chunk, 0)

  plsc.subcore_barrier()

  for k in range(1, KHOP + 1):
    def edge_pair(h, carry):
      for b in range(2):
        i = 2 * h + b

        @pl.when(h > 0)
        def _():
          pltpu.make_async_copy(rows[b], acc_sp.at[dstb[b]], ss[b]).wait()
        e0 = e_base + i * EC
        pltpu.sync_copy(src_hbm.at[pl.ds(e0, EC)], srcb[b])
        pltpu.sync_copy(dst_hbm.at[pl.ds(e0, EC)], dstb[b])
        for j in range(EC // L):
          sl = pl.ds(j * L, L)
          gidxb[b][sl] = srcb[b][sl] + cN
        pltpu.async_copy(y_hbm.at[gidxb[b]], rows[b], sg[b])
      for b in range(2):
        pltpu.make_async_copy(y_hbm.at[gidxb[b]], rows[b], sg[b]).wait()
        pltpu.async_copy(rows[b], acc_sp.at[dstb[b]], ss[b], add=True)
      return carry
    lax.fori_loop(0, NPAIR, edge_pair, 0)
    pltpu.make_async_copy(rows[0], acc_sp.at[dstb[0]], ss[0]).wait()
    e0 = e_base + (NEC - 1) * EC
    pltpu.sync_copy(src_hbm.at[pl.ds(e0, EC)], srcb[0])
    pltpu.sync_copy(dst_hbm.at[pl.ds(e0, EC)], dstb[0])
    for j in range(EC // L):
      sl = pl.ds(j * L, L)
      gidx0[sl] = src0[sl] + cN
    pltpu.async_copy(y_hbm.at[gidx0], rows0, sg0).wait()
    pltpu.async_copy(rows0, acc_sp.at[dstb[0]], ss[0], add=True)
    pltpu.make_async_copy(rows0, acc_sp.at[dstb[0]], ss[0]).wait()
    pltpu.make_async_copy(rows1, acc_sp.at[dstb[1]], ss[1]).wait()

    plsc.subcore_barrier()

    col0 = k * FD + c * HALF

    def pp_chunk(i, carry):
      r0 = (s + NS * i) * CH
      pltpu.sync_copy(acc_sp.at[pl.ds(r0, CH), :], accb)
      pltpu.sync_copy(y_hbm.at[pl.ds(cN + r0, CH), :], rows0)

      def rowf(r, rcarry):
        d = dinv_b[i * CH + r, :]
        for g in range(HALF // L):
          sl = pl.ds(g * L, L)
          x = d * (accb[r, sl] + rows0[r, sl])
          accb[r, sl] = x
          rows0[r, sl] = d * x
        return rcarry
      lax.fori_loop(0, CH, rowf, 0)

      pltpu.sync_copy(accb, out_hbm.at[pl.ds(r0, CH), pl.ds(col0, HALF)])
      pltpu.sync_copy(rows0, y_hbm.at[pl.ds(cN + r0, CH), :])
      zero_acc_chunk(r0)
      return carry
    lax.fori_loop(0, nchunks, pp_chunk, 0)

    plsc.subcore_barrier()


@jax.jit
def _lgcn(feature, src, dst):
  mesh = plsc.VectorSubcoreMesh(core_axis_name="c", subcore_axis_name="s")
  out, _ = pl.kernel(
      _body,
      out_type=(
          jax.ShapeDtypeStruct((N, (KHOP + 1) * FD), jnp.float32),
          jax.ShapeDtypeStruct((NC * N, HALF), jnp.float32),
      ),
      mesh=mesh,
      compiler_params=pltpu.CompilerParams(use_tc_tiling_on_sc=False),
      scratch_types=[
          pltpu.VMEM_SHARED((N, HALF), jnp.float32),
          pltpu.VMEM((CH * 8, L), jnp.float32),
          pltpu.VMEM((EC, HALF), jnp.float32),
          pltpu.VMEM((EC, HALF), jnp.float32),
          pltpu.VMEM((EC,), jnp.int32),
          pltpu.VMEM((EC,), jnp.int32),
          pltpu.VMEM((EC,), jnp.int32),
          pltpu.VMEM((EC,), jnp.int32),
          pltpu.VMEM((EC,), jnp.int32),
          pltpu.VMEM((EC,), jnp.int32),
          pltpu.VMEM((ZR, HALF), jnp.float32),
          pltpu.VMEM((CH, HALF), jnp.float32),
          pltpu.SemaphoreType.DMA,
          pltpu.SemaphoreType.DMA,
          pltpu.SemaphoreType.DMA,
          pltpu.SemaphoreType.DMA,
      ],
  )(feature, src, dst)
  return out


def kernel(feature, edge_index):
  ei = edge_index.astype(jnp.int32)
  return _lgcn(feature, ei[0], ei[1])


# speedup vs baseline: 9.6384x; 1.1486x over previous
import jax
import jax.numpy as jnp
from jax import lax
from jax.experimental import pallas as pl
from jax.experimental.pallas import tpu as pltpu
from jax.experimental.pallas import tpu_sc as plsc

N = 10000
FD = 256
HALF = 128
E = 160000
KHOP = 8
NS = 16
NC = 2
L = 16
CH = 80
NCH = N // CH
ZR = 16
EPT = E // NS
EC = 80
NEC = EPT // EC
G = 25
NG = NEC // G


def _body(f_hbm, src_hbm, dst_hbm, out_hbm, y_hbm,
          acc_sp,
          dinv_b, rows0, rows1, sbuf, dbuf, gbuf,
          zb, accb, sg0, sg1, ss0, ss1):
  c = lax.axis_index("c")
  s = lax.axis_index("s")
  cN = c * N
  e_base = s * EPT
  GE = G * EC

  rows = (rows0, rows1)
  sg = (sg0, sg1)
  ss = (ss0, ss1)

  fzero = jnp.zeros((L,), jnp.float32)
  fone = jnp.ones((L,), jnp.float32)

  def fill_zb(r, carry):
    for g in range(HALF // L):
      zb[r, pl.ds(g * L, L)] = fzero
    return carry
  lax.fori_loop(0, ZR, fill_zb, 0)

  def fill_ones(r, carry):
    for g in range(HALF // L):
      rows0[r, pl.ds(g * L, L)] = fone
    return carry
  lax.fori_loop(0, EC, fill_ones, 0)

  def zero_acc_chunk(r0):
    def zbody(z, carry):
      pltpu.sync_copy(zb, acc_sp.at[pl.ds(r0 + z * ZR, ZR), :])
      return carry
    lax.fori_loop(0, CH // ZR, zbody, 0)

  nchunks = (NCH - 1 - s) // NS + 1

  def zinit_body(i, carry):
    zero_acc_chunk((s + NS * i) * CH)
    return carry
  lax.fori_loop(0, nchunks, zinit_body, 0)

  plsc.subcore_barrier()

  def dslice(j):
    return dbuf.at[pl.ds(j * EC, EC)]

  def deg_group(g, carry):
    @pl.when(g > 0)
    def _():
      pltpu.make_async_copy(rows0, acc_sp.at[dslice(G - 2)], ss[1]).wait()
      pltpu.make_async_copy(rows0, acc_sp.at[dslice(G - 1)], ss[0]).wait()
    pltpu.sync_copy(dst_hbm.at[pl.ds(e_base + g * GE, GE)], dbuf)
    for j in range(G):
      b = j % 2
      if j >= 2:
        pltpu.make_async_copy(rows0, acc_sp.at[dslice(j - 2)], ss[b]).wait()
      pltpu.async_copy(rows0, acc_sp.at[dslice(j)], ss[b], add=True)
    return carry
  lax.fori_loop(0, NG, deg_group, 0)
  pltpu.make_async_copy(rows0, acc_sp.at[dslice(G - 2)], ss[1]).wait()
  pltpu.make_async_copy(rows0, acc_sp.at[dslice(G - 1)], ss[0]).wait()

  plsc.subcore_barrier()

  def dinv_chunk(i, carry):
    cid = s + NS * i
    pltpu.sync_copy(acc_sp.at[pl.ds(cid * CH, CH), :], accb)

    def dinv_body(r, rcarry):
      v = accb[r, pl.ds(0, L)] + 1.0
      g = jnp.full((L,), 1.0, jnp.float32)
      for j in range(1, 19):
        g = jnp.where(v >= float(1 << j),
                      jnp.full((L,), 2.0 ** (-0.5 * j), jnp.float32), g)
      for _ in range(5):
        g = g * (1.5 - 0.5 * v * g * g)
      dinv_b[i * CH + r, :] = g
      return rcarry
    lax.fori_loop(0, CH, dinv_body, 0)

    zero_acc_chunk(cid * CH)
    return carry
  lax.fori_loop(0, nchunks, dinv_chunk, 0)

  def p0_chunk(i, carry):
    r0 = (s + NS * i) * CH
    pltpu.sync_copy(f_hbm.at[pl.ds(r0, CH), pl.ds(c * HALF, HALF)], accb)

    def rowf(r, rcarry):
      d = dinv_b[i * CH + r, :]
      for g in range(HALF // L):
        sl = pl.ds(g * L, L)
        rows0[r, sl] = d * accb[r, sl]
      return rcarry
    lax.fori_loop(0, CH, rowf, 0)

    pltpu.sync_copy(accb, out_hbm.at[pl.ds(r0, CH), pl.ds(c * HALF, HALF)])
    pltpu.sync_copy(rows0, y_hbm.at[pl.ds(cN + r0, CH), :])
    return carry
  lax.fori_loop(0, nchunks, p0_chunk, 0)

  plsc.subcore_barrier()

  for k in range(1, KHOP + 1):
    def edge_group(g, carry):
      @pl.when(g > 0)
      def _():
        pltpu.make_async_copy(rows[1], acc_sp.at[dslice(G - 2)], ss[1]).wait()
        pltpu.make_async_copy(rows[0], acc_sp.at[dslice(G - 1)], ss[0]).wait()
      e0 = e_base + g * GE
      pltpu.sync_copy(src_hbm.at[pl.ds(e0, GE)], sbuf)
      pltpu.sync_copy(dst_hbm.at[pl.ds(e0, GE)], dbuf)

      def gidx_body(q, qcarry):
        sl = pl.ds(q * L, L)
        gbuf[sl] = sbuf[sl] + cN
        return qcarry
      lax.fori_loop(0, GE // L, gidx_body, 0)

      for j in range(G):
        b = j % 2
        if j >= 2:
          pltpu.make_async_copy(rows[b], acc_sp.at[dslice(j - 2)], ss[b]).wait()
        pltpu.async_copy(y_hbm.at[gbuf.at[pl.ds(j * EC, EC)]], rows[b],
                         sg[b]).wait()
        pltpu.async_copy(rows[b], acc_sp.at[dslice(j)], ss[b], add=True)
      return carry
    lax.fori_loop(0, NG, edge_group, 0)
    pltpu.make_async_copy(rows[1], acc_sp.at[dslice(G - 2)], ss[1]).wait()
    pltpu.make_async_copy(rows[0], acc_sp.at[dslice(G - 1)], ss[0]).wait()

    plsc.subcore_barrier()

    col0 = k * FD + c * HALF

    def pp_chunk(i, carry):
      r0 = (s + NS * i) * CH
      pltpu.sync_copy(acc_sp.at[pl.ds(r0, CH), :], accb)
      pltpu.sync_copy(y_hbm.at[pl.ds(cN + r0, CH), :], rows0)

      def rowf(r, rcarry):
        d = dinv_b[i * CH + r, :]
        for g in range(HALF // L):
          sl = pl.ds(g * L, L)
          x = d * (accb[r, sl] + rows0[r, sl])
          accb[r, sl] = x
          rows0[r, sl] = d * x
        return rcarry
      lax.fori_loop(0, CH, rowf, 0)

      pltpu.sync_copy(accb, out_hbm.at[pl.ds(r0, CH), pl.ds(col0, HALF)])
      pltpu.sync_copy(rows0, y_hbm.at[pl.ds(cN + r0, CH), :])
      zero_acc_chunk(r0)
      return carry
    lax.fori_loop(0, nchunks, pp_chunk, 0)

    plsc.subcore_barrier()


@jax.jit
def _lgcn(feature, src, dst):
  mesh = plsc.VectorSubcoreMesh(core_axis_name="c", subcore_axis_name="s")
  out, _ = pl.kernel(
      _body,
      out_type=(
          jax.ShapeDtypeStruct((N, (KHOP + 1) * FD), jnp.float32),
          jax.ShapeDtypeStruct((NC * N, HALF), jnp.float32),
      ),
      mesh=mesh,
      compiler_params=pltpu.CompilerParams(use_tc_tiling_on_sc=False),
      scratch_types=[
          pltpu.VMEM_SHARED((N, HALF), jnp.float32),
          pltpu.VMEM((CH * 8, L), jnp.float32),
          pltpu.VMEM((EC, HALF), jnp.float32),
          pltpu.VMEM((EC, HALF), jnp.float32),
          pltpu.VMEM((G * EC,), jnp.int32),
          pltpu.VMEM((G * EC,), jnp.int32),
          pltpu.VMEM((G * EC,), jnp.int32),
          pltpu.VMEM((ZR, HALF), jnp.float32),
          pltpu.VMEM((CH, HALF), jnp.float32),
          pltpu.SemaphoreType.DMA,
          pltpu.SemaphoreType.DMA,
          pltpu.SemaphoreType.DMA,
          pltpu.SemaphoreType.DMA,
      ],
  )(feature, src, dst)
  return out


def kernel(feature, edge_index):
  ei = edge_index.astype(jnp.int32)
  return _lgcn(feature, ei[0], ei[1])


# speedup vs baseline: 10.3202x; 1.0707x over previous
import jax
import jax.numpy as jnp
from jax import lax
from jax.experimental import pallas as pl
from jax.experimental.pallas import tpu as pltpu
from jax.experimental.pallas import tpu_sc as plsc

N = 10000
FD = 256
HALF = 128
E = 160000
KHOP = 8
NS = 16
NC = 2
L = 16
CH = 80
NCH = N // CH
MAXC = 8
ZR = 20
SH = 40
EPT = E // NS
EC = 80
NEC = EPT // EC
G = 25
NG = NEC // G


def _body(f_hbm, src_hbm, dst_hbm, out_hbm, y_hbm,
          acc_sp,
          dinv_b, rows0, rows1, sbuf, dbuf, gbuf,
          zb, accb, sg0, sg1, ss0, ss1,
          sr0, sr1, sr2, so0, so1, so2, sy0, sy1, sy2, zsem):
  c = lax.axis_index("c")
  s = lax.axis_index("s")
  cN = c * N
  e_base = s * EPT
  GE = G * EC

  rows = (rows0, rows1)
  sg = (sg0, sg1)
  ss = (ss0, ss1)
  sr = (sr0, sr1, sr2)
  so = (so0, so1, so2)
  sy = (sy0, sy1, sy2)
  ACCH = ((accb, 0), (accb, SH), (rows1, 0))
  YH = ((rows0, 0), (rows0, SH), (rows1, SH))

  fzero = jnp.zeros((L,), jnp.float32)
  fone = jnp.ones((L,), jnp.float32)
  iota = lax.iota(jnp.int32, L)

  def fill_zb(r, carry):
    for g in range(HALF // L):
      zb[r, pl.ds(g * L, L)] = fzero
    return carry
  lax.fori_loop(0, ZR, fill_zb, 0)

  def fill_ones(r, carry):
    for g in range(HALF // L):
      rows0[r, pl.ds(g * L, L)] = fone
    return carry
  lax.fori_loop(0, EC, fill_ones, 0)

  def zero_acc_chunk(r0):
    def zbody(z, carry):
      pltpu.sync_copy(zb, acc_sp.at[pl.ds(r0 + z * ZR, ZR), :])
      return carry
    lax.fori_loop(0, CH // ZR, zbody, 0)

  nchunks = (NCH - 1 - s) // NS + 1

  def zinit_body(i, carry):
    zero_acc_chunk((s + NS * i) * CH)
    return carry
  lax.fori_loop(0, nchunks, zinit_body, 0)

  plsc.subcore_barrier()

  def dslice(j):
    return dbuf.at[pl.ds(j * EC, EC)]

  def deg_group(g, carry):
    @pl.when(g > 0)
    def _():
      pltpu.make_async_copy(rows0, acc_sp.at[dslice(G - 2)], ss[1]).wait()
      pltpu.make_async_copy(rows0, acc_sp.at[dslice(G - 1)], ss[0]).wait()
    pltpu.sync_copy(dst_hbm.at[pl.ds(e_base + g * GE, GE)], dbuf)
    for j in range(G):
      b = j % 2
      if j >= 2:
        pltpu.make_async_copy(rows0, acc_sp.at[dslice(j - 2)], ss[b]).wait()
      pltpu.async_copy(rows0, acc_sp.at[dslice(j)], ss[b], add=True)
    return carry
  lax.fori_loop(0, NG, deg_group, 0)
  pltpu.make_async_copy(rows0, acc_sp.at[dslice(G - 2)], ss[1]).wait()
  pltpu.make_async_copy(rows0, acc_sp.at[dslice(G - 1)], ss[0]).wait()

  plsc.subcore_barrier()

  def dinv_chunk(i, carry):
    cid = s + NS * i
    pltpu.sync_copy(acc_sp.at[pl.ds(cid * CH, CH), :], accb)

    def dinv_body(r, rcarry):
      v = accb[r, pl.ds(0, L)] + 1.0
      g = jnp.full((L,), 1.0, jnp.float32)
      for j in range(1, 19):
        g = jnp.where(v >= float(1 << j),
                      jnp.full((L,), 2.0 ** (-0.5 * j), jnp.float32), g)
      for _ in range(5):
        g = g * (1.5 - 0.5 * v * g * g)
      dinv_b[i * CH + r, :] = g
      return rcarry
    lax.fori_loop(0, CH, dinv_body, 0)

    zero_acc_chunk(cid * CH)
    return carry
  lax.fori_loop(0, nchunks, dinv_chunk, 0)

  def p0_chunk(i, carry):
    r0 = (s + NS * i) * CH
    pltpu.sync_copy(f_hbm.at[pl.ds(r0, CH), pl.ds(c * HALF, HALF)], accb)

    def rowf(r, rcarry):
      d = dinv_b[i * CH + r, :]
      for g in range(HALF // L):
        sl = pl.ds(g * L, L)
        rows0[r, sl] = d * accb[r, sl]
      return rcarry
    lax.fori_loop(0, CH, rowf, 0)

    pltpu.sync_copy(accb, out_hbm.at[pl.ds(r0, CH), pl.ds(c * HALF, HALF)])
    pltpu.sync_copy(rows0, y_hbm.at[pl.ds(cN + r0, CH), :])
    return carry
  lax.fori_loop(0, nchunks, p0_chunk, 0)

  plsc.subcore_barrier()

  for k in range(1, KHOP + 1):
    def edge_group(g, carry):
      @pl.when(g > 0)
      def _():
        pltpu.make_async_copy(rows[1], acc_sp.at[dslice(G - 2)], ss[1]).wait()
        pltpu.make_async_copy(rows[0], acc_sp.at[dslice(G - 1)], ss[0]).wait()
      e0 = e_base + g * GE
      pltpu.sync_copy(src_hbm.at[pl.ds(e0, GE)], sbuf)
      pltpu.sync_copy(dst_hbm.at[pl.ds(e0, GE)], dbuf)

      def gidx_body(q, qcarry):
        sl = pl.ds(q * L, L)
        gbuf[sl] = sbuf[sl] + cN
        return qcarry
      lax.fori_loop(0, GE // L, gidx_body, 0)

      for j in range(G):
        b = j % 2
        if j >= 2:
          pltpu.make_async_copy(rows[b], acc_sp.at[dslice(j - 2)], ss[b]).wait()
        pltpu.async_copy(y_hbm.at[gbuf.at[pl.ds(j * EC, EC)]], rows[b],
                         sg[b]).wait()
        pltpu.async_copy(rows[b], acc_sp.at[dslice(j)], ss[b], add=True)
      return carry
    lax.fori_loop(0, NG, edge_group, 0)
    pltpu.make_async_copy(rows[1], acc_sp.at[dslice(G - 2)], ss[1]).wait()
    pltpu.make_async_copy(rows[0], acc_sp.at[dslice(G - 1)], ss[0]).wait()

    def self_pair(h, carry):
      for b in range(2):
        i = 2 * h + b

        @pl.when(i < nchunks)
        def _():
          r0 = (s + NS * i) * CH

          @pl.when(h > 0)
          def _():
            pltpu.make_async_copy(rows[b], acc_sp.at[dslice(b)], ss[b]).wait()

          def sidx(q, qcarry):
            dbuf[pl.ds(b * EC + q * L, L)] = r0 + q * L + iota
            return qcarry
          lax.fori_loop(0, EC // L, sidx, 0)
          pltpu.async_copy(y_hbm.at[pl.ds(cN + r0, CH), :], rows[b],
                           sg[b]).wait()
          pltpu.async_copy(rows[b], acc_sp.at[dslice(b)], ss[b], add=True)
      return carry
    lax.fori_loop(0, (MAXC + 1) // 2, self_pair, 0)
    pltpu.make_async_copy(rows[0], acc_sp.at[dslice(0)], ss[0]).wait()
    pltpu.make_async_copy(rows[1], acc_sp.at[dslice(1)], ss[1]).wait()

    plsc.subcore_barrier()

    col0 = k * FD + c * HALF
    ns2 = 2 * nchunks

    def stage_r0(m):
      return (s + NS * (m // 2)) * CH + (m % 2) * SH

    def issue_read(m, t):
      ab, off = ACCH[t]
      pltpu.async_copy(acc_sp.at[pl.ds(stage_r0(m), SH), :],
                       ab.at[pl.ds(off, SH), :], sr[t])

    issue_read(0, 0)

    def pp_triple(h, carry):
      for t in range(3):
        m = 3 * h + t

        @pl.when(m < ns2)
        def _():
          ab, aoff = ACCH[t]
          yb_, yoff = YH[t]
          t1 = (t + 1) % 3
          nab, naoff = ACCH[t1]
          r0 = stage_r0(m)

          @pl.when(m + 1 < ns2)
          def _():
            @pl.when(m + 1 >= 3)
            def _():
              pltpu.make_async_copy(
                  nab.at[pl.ds(naoff, SH), :],
                  out_hbm.at[pl.ds(0, SH), pl.ds(c * HALF, HALF)],
                  so[t1]).wait()
            issue_read(m + 1, t1)

          pltpu.make_async_copy(acc_sp.at[pl.ds(0, SH), :],
                                ab.at[pl.ds(aoff, SH), :], sr[t]).wait()

          @pl.when(m >= 3)
          def _():
            pltpu.make_async_copy(yb_.at[pl.ds(yoff, SH), :],
                                  y_hbm.at[pl.ds(cN, SH), :], sy[t]).wait()

          drow = (m // 2) * CH + (m % 2) * SH

          def rowf(r, rcarry):
            d = dinv_b[drow + r, :]
            for gq in range(HALF // L):
              sl = pl.ds(gq * L, L)
              x = d * ab[aoff + r, sl]
              ab[aoff + r, sl] = x
              yb_[yoff + r, sl] = d * x
            return rcarry
          lax.fori_loop(0, SH, rowf, 0)

          pltpu.async_copy(ab.at[pl.ds(aoff, SH), :],
                           out_hbm.at[pl.ds(r0, SH), pl.ds(col0, HALF)],
                           so[t])
          pltpu.async_copy(yb_.at[pl.ds(yoff, SH), :],
                           y_hbm.at[pl.ds(cN + r0, SH), :], sy[t])
          for z in range(SH // ZR):
            pltpu.async_copy(zb, acc_sp.at[pl.ds(r0 + z * ZR, ZR), :], zsem)
      return carry
    lax.fori_loop(0, (2 * MAXC + 2) // 3, pp_triple, 0)

    for t in range(3):
      ab, aoff = ACCH[t]
      yb_, yoff = YH[t]
      pltpu.make_async_copy(ab.at[pl.ds(aoff, SH), :],
                            out_hbm.at[pl.ds(0, SH), pl.ds(c * HALF, HALF)],
                            so[t]).wait()
      pltpu.make_async_copy(yb_.at[pl.ds(yoff, SH), :],
                            y_hbm.at[pl.ds(cN, SH), :], sy[t]).wait()

    def zdrain(q, carry):
      pltpu.make_async_copy(zb, acc_sp.at[pl.ds(0, ZR), :], zsem).wait()
      return carry
    lax.fori_loop(0, 2 * ns2, zdrain, 0)

    plsc.subcore_barrier()


@jax.jit
def _lgcn(feature, src, dst):
  mesh = plsc.VectorSubcoreMesh(core_axis_name="c", subcore_axis_name="s")
  out, _ = pl.kernel(
      _body,
      out_type=(
          jax.ShapeDtypeStruct((N, (KHOP + 1) * FD), jnp.float32),
          jax.ShapeDtypeStruct((NC * N, HALF), jnp.float32),
      ),
      mesh=mesh,
      compiler_params=pltpu.CompilerParams(use_tc_tiling_on_sc=False),
      scratch_types=[
          pltpu.VMEM_SHARED((N, HALF), jnp.float32),
          pltpu.VMEM((CH * 8, L), jnp.float32),
          pltpu.VMEM((EC, HALF), jnp.float32),
          pltpu.VMEM((EC, HALF), jnp.float32),
          pltpu.VMEM((G * EC,), jnp.int32),
          pltpu.VMEM((G * EC,), jnp.int32),
          pltpu.VMEM((G * EC,), jnp.int32),
          pltpu.VMEM((ZR, HALF), jnp.float32),
          pltpu.VMEM((CH, HALF), jnp.float32),
          pltpu.SemaphoreType.DMA,
          pltpu.SemaphoreType.DMA,
          pltpu.SemaphoreType.DMA,
          pltpu.SemaphoreType.DMA,
          pltpu.SemaphoreType.DMA,
          pltpu.SemaphoreType.DMA,
          pltpu.SemaphoreType.DMA,
          pltpu.SemaphoreType.DMA,
          pltpu.SemaphoreType.DMA,
          pltpu.SemaphoreType.DMA,
          pltpu.SemaphoreType.DMA,
          pltpu.SemaphoreType.DMA,
          pltpu.SemaphoreType.DMA,
          pltpu.SemaphoreType.DMA,
      ],
  )(feature, src, dst)
  return out


def kernel(feature, edge_index):
  ei = edge_index.astype(jnp.int32)
  return _lgcn(feature, ei[0], ei[1])


# speedup vs baseline: 14.1273x; 1.3689x over previous
import jax
import jax.numpy as jnp
from jax import lax
from jax.experimental import pallas as pl
from jax.experimental.pallas import tpu as pltpu
from jax.experimental.pallas import tpu_sc as plsc

N = 10000
FD = 256
HALF = 128
E = 160000
KHOP = 8
NS = 16
NC = 2
L = 16
CH = 80
NCH = N // CH
MAXC = 8
ZR = 20
SH = 40
EPT = E // NS
EC = 80
NEC = EPT // EC
G = 25
NG = NEC // G


def _body(f_hbm, src_hbm, dst_hbm, out_hbm, y_hbm,
          acc_sp,
          dinv_b, rows0, rows1, sbuf, dbuf, gbuf,
          zb, accb, sg0, sg1, sg2, ss0, ss1, ss2,
          sr0, sr1, sr2, so0, so1, so2, sy0, sy1, sy2, zsem):
  c = lax.axis_index("c")
  s = lax.axis_index("s")
  cN = c * N
  e_base = s * EPT
  GE = G * EC

  rows = (rows0, rows1)
  sg = (sg0, sg1)
  ss = (ss0, ss1)
  ROWS3 = (rows0, rows1, accb)
  sg3 = (sg0, sg1, sg2)
  ss3 = (ss0, ss1, ss2)
  sr = (sr0, sr1, sr2)
  so = (so0, so1, so2)
  sy = (sy0, sy1, sy2)
  ACCH = ((accb, 0), (accb, SH), (rows1, 0))
  YH = ((rows0, 0), (rows0, SH), (rows1, SH))

  fzero = jnp.zeros((L,), jnp.float32)
  fone = jnp.ones((L,), jnp.float32)
  iota = lax.iota(jnp.int32, L)

  def fill_zb(r, carry):
    for g in range(HALF // L):
      zb[r, pl.ds(g * L, L)] = fzero
    return carry
  lax.fori_loop(0, ZR, fill_zb, 0)

  def fill_ones(r, carry):
    for g in range(HALF // L):
      rows0[r, pl.ds(g * L, L)] = fone
    return carry
  lax.fori_loop(0, EC, fill_ones, 0)

  def zero_acc_chunk(r0):
    def zbody(z, carry):
      pltpu.sync_copy(zb, acc_sp.at[pl.ds(r0 + z * ZR, ZR), :])
      return carry
    lax.fori_loop(0, CH // ZR, zbody, 0)

  nchunks = (NCH - 1 - s) // NS + 1

  def zinit_body(i, carry):
    zero_acc_chunk((s + NS * i) * CH)
    return carry
  lax.fori_loop(0, nchunks, zinit_body, 0)

  plsc.subcore_barrier()

  def dslice(j):
    return dbuf.at[pl.ds(j * EC, EC)]

  def deg_group(g, carry):
    @pl.when(g > 0)
    def _():
      pltpu.make_async_copy(rows0, acc_sp.at[dslice(G - 2)], ss[1]).wait()
      pltpu.make_async_copy(rows0, acc_sp.at[dslice(G - 1)], ss[0]).wait()
    pltpu.sync_copy(dst_hbm.at[pl.ds(e_base + g * GE, GE)], dbuf)
    for j in range(G):
      b = j % 2
      if j >= 2:
        pltpu.make_async_copy(rows0, acc_sp.at[dslice(j - 2)], ss[b]).wait()
      pltpu.async_copy(rows0, acc_sp.at[dslice(j)], ss[b], add=True)
    return carry
  lax.fori_loop(0, NG, deg_group, 0)
  pltpu.make_async_copy(rows0, acc_sp.at[dslice(G - 2)], ss[1]).wait()
  pltpu.make_async_copy(rows0, acc_sp.at[dslice(G - 1)], ss[0]).wait()

  plsc.subcore_barrier()

  def dinv_chunk(i, carry):
    cid = s + NS * i
    pltpu.sync_copy(acc_sp.at[pl.ds(cid * CH, CH), :], accb)

    def dinv_body(r, rcarry):
      v = accb[r, pl.ds(0, L)] + 1.0
      g = jnp.full((L,), 1.0, jnp.float32)
      for j in range(1, 19):
        g = jnp.where(v >= float(1 << j),
                      jnp.full((L,), 2.0 ** (-0.5 * j), jnp.float32), g)
      for _ in range(5):
        g = g * (1.5 - 0.5 * v * g * g)
      dinv_b[i * CH + r, :] = g
      return rcarry
    lax.fori_loop(0, CH, dinv_body, 0)

    zero_acc_chunk(cid * CH)
    return carry
  lax.fori_loop(0, nchunks, dinv_chunk, 0)

  def p0_chunk(i, carry):
    r0 = (s + NS * i) * CH
    pltpu.sync_copy(f_hbm.at[pl.ds(r0, CH), pl.ds(c * HALF, HALF)], accb)

    def rowf(r, rcarry):
      d = dinv_b[i * CH + r, :]
      for g in range(HALF // L):
        sl = pl.ds(g * L, L)
        rows0[r, sl] = d * accb[r, sl]
      return rcarry
    lax.fori_loop(0, CH, rowf, 0)

    pltpu.sync_copy(accb, out_hbm.at[pl.ds(r0, CH), pl.ds(c * HALF, HALF)])
    pltpu.sync_copy(rows0, y_hbm.at[pl.ds(cN + r0, CH), :])
    return carry
  lax.fori_loop(0, nchunks, p0_chunk, 0)

  plsc.subcore_barrier()

  for k in range(1, KHOP + 1):
    def gslice(j):
      return gbuf.at[pl.ds(j * EC, EC)]

    def drain3(tag):
      for jj in (G - 3, G - 2, G - 1):
        t = jj % 3
        pltpu.make_async_copy(ROWS3[t], acc_sp.at[dslice(jj)],
                              ss3[t]).wait()

    def edge_group(g, carry):
      @pl.when(g > 0)
      def _():
        drain3(0)
      e0 = e_base + g * GE
      pltpu.sync_copy(src_hbm.at[pl.ds(e0, GE)], sbuf)
      pltpu.sync_copy(dst_hbm.at[pl.ds(e0, GE)], dbuf)

      def gidx_body(q, qcarry):
        sl = pl.ds(q * L, L)
        gbuf[sl] = sbuf[sl] + cN
        return qcarry
      lax.fori_loop(0, GE // L, gidx_body, 0)

      pltpu.async_copy(y_hbm.at[gslice(0)], ROWS3[0], sg3[0])
      for j in range(1, G):
        t = j % 3
        tp = (j - 1) % 3
        if j >= 3:
          pltpu.make_async_copy(ROWS3[t], acc_sp.at[dslice(j - 3)],
                                ss3[t]).wait()
        pltpu.async_copy(y_hbm.at[gslice(j)], ROWS3[t], sg3[t])
        pltpu.make_async_copy(y_hbm.at[gslice(j - 1)], ROWS3[tp],
                              sg3[tp]).wait()
        pltpu.async_copy(ROWS3[tp], acc_sp.at[dslice(j - 1)], ss3[tp],
                         add=True)
      tl = (G - 1) % 3
      pltpu.make_async_copy(y_hbm.at[gslice(G - 1)], ROWS3[tl],
                            sg3[tl]).wait()
      pltpu.async_copy(ROWS3[tl], acc_sp.at[dslice(G - 1)], ss3[tl],
                       add=True)
      return carry
    lax.fori_loop(0, NG, edge_group, 0)
    drain3(1)

    def self_pair(h, carry):
      for b in range(2):
        i = 2 * h + b

        @pl.when(i < nchunks)
        def _():
          r0 = (s + NS * i) * CH

          @pl.when(h > 0)
          def _():
            pltpu.make_async_copy(rows[b], acc_sp.at[dslice(b)], ss[b]).wait()

          def sidx(q, qcarry):
            dbuf[pl.ds(b * EC + q * L, L)] = r0 + q * L + iota
            return qcarry
          lax.fori_loop(0, EC // L, sidx, 0)
          pltpu.async_copy(y_hbm.at[pl.ds(cN + r0, CH), :], rows[b],
                           sg[b]).wait()
          pltpu.async_copy(rows[b], acc_sp.at[dslice(b)], ss[b], add=True)
      return carry
    lax.fori_loop(0, (MAXC + 1) // 2, self_pair, 0)
    pltpu.make_async_copy(rows[0], acc_sp.at[dslice(0)], ss[0]).wait()
    pltpu.make_async_copy(rows[1], acc_sp.at[dslice(1)], ss[1]).wait()

    plsc.subcore_barrier()

    col0 = k * FD + c * HALF
    ns2 = 2 * nchunks

    def stage_r0(m):
      return (s + NS * (m // 2)) * CH + (m % 2) * SH

    def issue_read(m, t):
      ab, off = ACCH[t]
      pltpu.async_copy(acc_sp.at[pl.ds(stage_r0(m), SH), :],
                       ab.at[pl.ds(off, SH), :], sr[t])

    issue_read(0, 0)

    def pp_triple(h, carry):
      for t in range(3):
        m = 3 * h + t

        @pl.when(m < ns2)
        def _():
          ab, aoff = ACCH[t]
          yb_, yoff = YH[t]
          t1 = (t + 1) % 3
          nab, naoff = ACCH[t1]
          r0 = stage_r0(m)

          @pl.when(m + 1 < ns2)
          def _():
            @pl.when(m + 1 >= 3)
            def _():
              pltpu.make_async_copy(
                  nab.at[pl.ds(naoff, SH), :],
                  out_hbm.at[pl.ds(0, SH), pl.ds(c * HALF, HALF)],
                  so[t1]).wait()
            issue_read(m + 1, t1)

          pltpu.make_async_copy(acc_sp.at[pl.ds(0, SH), :],
                                ab.at[pl.ds(aoff, SH), :], sr[t]).wait()

          @pl.when(m >= 3)
          def _():
            pltpu.make_async_copy(yb_.at[pl.ds(yoff, SH), :],
                                  y_hbm.at[pl.ds(cN, SH), :], sy[t]).wait()

          drow = (m // 2) * CH + (m % 2) * SH

          def rowf(r, rcarry):
            d = dinv_b[drow + r, :]
            for gq in range(HALF // L):
              sl = pl.ds(gq * L, L)
              x = d * ab[aoff + r, sl]
              ab[aoff + r, sl] = x
              yb_[yoff + r, sl] = d * x
            return rcarry
          lax.fori_loop(0, SH, rowf, 0)

          pltpu.async_copy(ab.at[pl.ds(aoff, SH), :],
                           out_hbm.at[pl.ds(r0, SH), pl.ds(col0, HALF)],
                           so[t])
          pltpu.async_copy(yb_.at[pl.ds(yoff, SH), :],
                           y_hbm.at[pl.ds(cN + r0, SH), :], sy[t])
          for z in range(SH // ZR):
            pltpu.async_copy(zb, acc_sp.at[pl.ds(r0 + z * ZR, ZR), :], zsem)
      return carry
    lax.fori_loop(0, (2 * MAXC + 2) // 3, pp_triple, 0)

    for t in range(3):
      ab, aoff = ACCH[t]
      yb_, yoff = YH[t]
      pltpu.make_async_copy(ab.at[pl.ds(aoff, SH), :],
                            out_hbm.at[pl.ds(0, SH), pl.ds(c * HALF, HALF)],
                            so[t]).wait()
      pltpu.make_async_copy(yb_.at[pl.ds(yoff, SH), :],
                            y_hbm.at[pl.ds(cN, SH), :], sy[t]).wait()

    def zdrain(q, carry):
      pltpu.make_async_copy(zb, acc_sp.at[pl.ds(0, ZR), :], zsem).wait()
      return carry
    lax.fori_loop(0, 2 * ns2, zdrain, 0)

    plsc.subcore_barrier()


@jax.jit
def _lgcn(feature, src, dst):
  mesh = plsc.VectorSubcoreMesh(core_axis_name="c", subcore_axis_name="s")
  out, _ = pl.kernel(
      _body,
      out_type=(
          jax.ShapeDtypeStruct((N, (KHOP + 1) * FD), jnp.float32),
          jax.ShapeDtypeStruct((NC * N, HALF), jnp.float32),
      ),
      mesh=mesh,
      compiler_params=pltpu.CompilerParams(use_tc_tiling_on_sc=False),
      scratch_types=[
          pltpu.VMEM_SHARED((N, HALF), jnp.float32),
          pltpu.VMEM((CH * 8, L), jnp.float32),
          pltpu.VMEM((EC, HALF), jnp.float32),
          pltpu.VMEM((EC, HALF), jnp.float32),
          pltpu.VMEM((G * EC,), jnp.int32),
          pltpu.VMEM((G * EC,), jnp.int32),
          pltpu.VMEM((G * EC,), jnp.int32),
          pltpu.VMEM((ZR, HALF), jnp.float32),
          pltpu.VMEM((CH, HALF), jnp.float32),
          pltpu.SemaphoreType.DMA,
          pltpu.SemaphoreType.DMA,
          pltpu.SemaphoreType.DMA,
          pltpu.SemaphoreType.DMA,
          pltpu.SemaphoreType.DMA,
          pltpu.SemaphoreType.DMA,
          pltpu.SemaphoreType.DMA,
          pltpu.SemaphoreType.DMA,
          pltpu.SemaphoreType.DMA,
          pltpu.SemaphoreType.DMA,
          pltpu.SemaphoreType.DMA,
          pltpu.SemaphoreType.DMA,
          pltpu.SemaphoreType.DMA,
          pltpu.SemaphoreType.DMA,
          pltpu.SemaphoreType.DMA,
          pltpu.SemaphoreType.DMA,
      ],
  )(feature, src, dst)
  return out


def kernel(feature, edge_index):
  ei = edge_index.astype(jnp.int32)
  return _lgcn(feature, ei[0], ei[1])


# speedup vs baseline: 14.3144x; 1.0132x over previous
import jax
import jax.numpy as jnp
from jax import lax
from jax.experimental import pallas as pl
from jax.experimental.pallas import tpu as pltpu
from jax.experimental.pallas import tpu_sc as plsc

N = 10000
FD = 256
HALF = 128
E = 160000
KHOP = 8
NS = 16
NC = 2
L = 16
CH = 80
NCH = N // CH
MAXC = 8
ZR = 20
SH = 40
EPT = E // NS
EC = 80
NEC = EPT // EC
G = 25
NG = NEC // G


def _body(f_hbm, src_hbm, dst_hbm, out_hbm, y_hbm,
          acc_sp,
          dinv_b, rows0, rows1, sbuf, dbuf, gbuf,
          zb, accb, sg0, sg1, sg2, ss0, ss1, ss2,
          sr0, sr1, sr2, so0, so1, so2, sy0, sy1, sy2, zsem):
  c = lax.axis_index("c")
  s = lax.axis_index("s")
  cN = c * N
  e_base = s * EPT
  GE = G * EC

  rows = (rows0, rows1)
  sg = (sg0, sg1)
  ss = (ss0, ss1)
  ROWS3 = (rows0, rows1, accb)
  sg3 = (sg0, sg1, sg2)
  ss3 = (ss0, ss1, ss2)
  sr = (sr0, sr1, sr2)
  so = (so0, so1, so2)
  sy = (sy0, sy1, sy2)
  ACCH = ((accb, 0), (accb, SH), (rows1, 0))
  YH = ((rows0, 0), (rows0, SH), (rows1, SH))

  fzero = jnp.zeros((L,), jnp.float32)
  fone = jnp.ones((L,), jnp.float32)
  iota = lax.iota(jnp.int32, L)

  def fill_zb(r, carry):
    for g in range(HALF // L):
      zb[r, pl.ds(g * L, L)] = fzero
    return carry
  lax.fori_loop(0, ZR, fill_zb, 0)

  def fill_ones(r, carry):
    for g in range(HALF // L):
      rows0[r, pl.ds(g * L, L)] = fone
    return carry
  lax.fori_loop(0, EC, fill_ones, 0)

  def zero_acc_chunk(r0):
    def zbody(z, carry):
      pltpu.sync_copy(zb, acc_sp.at[pl.ds(r0 + z * ZR, ZR), :])
      return carry
    lax.fori_loop(0, CH // ZR, zbody, 0)

  nchunks = (NCH - 1 - s) // NS + 1

  def zinit_body(i, carry):
    zero_acc_chunk((s + NS * i) * CH)
    return carry
  lax.fori_loop(0, nchunks, zinit_body, 0)

  plsc.subcore_barrier()

  def dslice(j):
    return dbuf.at[pl.ds(j * EC, EC)]

  def deg_drain3():
    for jj in (G - 3, G - 2, G - 1):
      pltpu.make_async_copy(rows0, acc_sp.at[dslice(jj)],
                            ss3[jj % 3]).wait()

  def deg_group(g, carry):
    @pl.when(g > 0)
    def _():
      deg_drain3()
    pltpu.sync_copy(dst_hbm.at[pl.ds(e_base + g * GE, GE)], dbuf)
    for j in range(G):
      t = j % 3
      if j >= 3:
        pltpu.make_async_copy(rows0, acc_sp.at[dslice(j - 3)],
                              ss3[t]).wait()
      pltpu.async_copy(rows0, acc_sp.at[dslice(j)], ss3[t], add=True)
    return carry
  lax.fori_loop(0, NG, deg_group, 0)
  deg_drain3()

  plsc.subcore_barrier()

  def dinv_chunk(i, carry):
    cid = s + NS * i
    pltpu.sync_copy(acc_sp.at[pl.ds(cid * CH, CH), :], accb)

    def dinv_body(r, rcarry):
      v = accb[r, pl.ds(0, L)] + 1.0
      g = jnp.full((L,), 1.0, jnp.float32)
      for j in range(1, 19):
        g = jnp.where(v >= float(1 << j),
                      jnp.full((L,), 2.0 ** (-0.5 * j), jnp.float32), g)
      for _ in range(5):
        g = g * (1.5 - 0.5 * v * g * g)
      dinv_b[i * CH + r, :] = g
      return rcarry
    lax.fori_loop(0, CH, dinv_body, 0)

    zero_acc_chunk(cid * CH)
    return carry
  lax.fori_loop(0, nchunks, dinv_chunk, 0)

  def p0_chunk(i, carry):
    r0 = (s + NS * i) * CH
    pltpu.sync_copy(f_hbm.at[pl.ds(r0, CH), pl.ds(c * HALF, HALF)], accb)

    @plsc.parallel_loop(0, CH, unroll=2)
    def _(r):
      d = dinv_b[i * CH + r, :]
      for g in range(HALF // L):
        sl = pl.ds(g * L, L)
        rows0[r, sl] = d * accb[r, sl]

    pltpu.sync_copy(accb, out_hbm.at[pl.ds(r0, CH), pl.ds(c * HALF, HALF)])
    pltpu.sync_copy(rows0, y_hbm.at[pl.ds(cN + r0, CH), :])
    return carry
  lax.fori_loop(0, nchunks, p0_chunk, 0)

  plsc.subcore_barrier()

  for k in range(1, KHOP + 1):
    def gslice(j):
      return gbuf.at[pl.ds(j * EC, EC)]

    def drain3(tag):
      for jj in (G - 3, G - 2, G - 1):
        t = jj % 3
        pltpu.make_async_copy(ROWS3[t], acc_sp.at[dslice(jj)],
                              ss3[t]).wait()

    def edge_group(g, carry):
      @pl.when(g > 0)
      def _():
        drain3(0)
      e0 = e_base + g * GE
      pltpu.sync_copy(src_hbm.at[pl.ds(e0, GE)], sbuf)
      pltpu.sync_copy(dst_hbm.at[pl.ds(e0, GE)], dbuf)

      @plsc.parallel_loop(0, GE // L, unroll=4)
      def _(q):
        sl = pl.ds(q * L, L)
        gbuf[sl] = sbuf[sl] + cN

      pltpu.async_copy(y_hbm.at[gslice(0)], ROWS3[0], sg3[0])
      for j in range(1, G):
        t = j % 3
        tp = (j - 1) % 3
        if j >= 3:
          pltpu.make_async_copy(ROWS3[t], acc_sp.at[dslice(j - 3)],
                                ss3[t]).wait()
        pltpu.async_copy(y_hbm.at[gslice(j)], ROWS3[t], sg3[t])
        pltpu.make_async_copy(y_hbm.at[gslice(j - 1)], ROWS3[tp],
                              sg3[tp]).wait()
        pltpu.async_copy(ROWS3[tp], acc_sp.at[dslice(j - 1)], ss3[tp],
                         add=True)
      tl = (G - 1) % 3
      pltpu.make_async_copy(y_hbm.at[gslice(G - 1)], ROWS3[tl],
                            sg3[tl]).wait()
      pltpu.async_copy(ROWS3[tl], acc_sp.at[dslice(G - 1)], ss3[tl],
                       add=True)
      return carry
    lax.fori_loop(0, NG, edge_group, 0)
    drain3(1)

    def self_pair(h, carry):
      for b in range(2):
        i = 2 * h + b

        @pl.when(i < nchunks)
        def _():
          r0 = (s + NS * i) * CH

          @pl.when(h > 0)
          def _():
            pltpu.make_async_copy(rows[b], acc_sp.at[dslice(b)], ss[b]).wait()

          def sidx(q, qcarry):
            dbuf[pl.ds(b * EC + q * L, L)] = r0 + q * L + iota
            return qcarry
          lax.fori_loop(0, EC // L, sidx, 0)
          pltpu.async_copy(y_hbm.at[pl.ds(cN + r0, CH), :], rows[b],
                           sg[b]).wait()
          pltpu.async_copy(rows[b], acc_sp.at[dslice(b)], ss[b], add=True)
      return carry
    lax.fori_loop(0, (MAXC + 1) // 2, self_pair, 0)
    pltpu.make_async_copy(rows[0], acc_sp.at[dslice(0)], ss[0]).wait()
    pltpu.make_async_copy(rows[1], acc_sp.at[dslice(1)], ss[1]).wait()

    plsc.subcore_barrier()

    col0 = k * FD + c * HALF
    ns2 = 2 * nchunks

    def stage_r0(m):
      return (s + NS * (m // 2)) * CH + (m % 2) * SH

    def issue_read(m, t):
      ab, off = ACCH[t]
      pltpu.async_copy(acc_sp.at[pl.ds(stage_r0(m), SH), :],
                       ab.at[pl.ds(off, SH), :], sr[t])

    issue_read(0, 0)

    def pp_triple(h, carry):
      for t in range(3):
        m = 3 * h + t

        @pl.when(m < ns2)
        def _():
          ab, aoff = ACCH[t]
          yb_, yoff = YH[t]
          t1 = (t + 1) % 3
          nab, naoff = ACCH[t1]
          r0 = stage_r0(m)

          @pl.when(m + 1 < ns2)
          def _():
            @pl.when(m + 1 >= 3)
            def _():
              pltpu.make_async_copy(
                  nab.at[pl.ds(naoff, SH), :],
                  out_hbm.at[pl.ds(0, SH), pl.ds(c * HALF, HALF)],
                  so[t1]).wait()
            issue_read(m + 1, t1)

          pltpu.make_async_copy(acc_sp.at[pl.ds(0, SH), :],
                                ab.at[pl.ds(aoff, SH), :], sr[t]).wait()

          if k < KHOP:
            @pl.when(m >= 3)
            def _():
              pltpu.make_async_copy(yb_.at[pl.ds(yoff, SH), :],
                                    y_hbm.at[pl.ds(cN, SH), :],
                                    sy[t]).wait()

          drow = (m // 2) * CH + (m % 2) * SH

          if k < KHOP:
            @plsc.parallel_loop(0, SH, unroll=2)
            def _(r):
              d = dinv_b[drow + r, :]
              for gq in range(HALF // L):
                sl = pl.ds(gq * L, L)
                x = d * ab[aoff + r, sl]
                ab[aoff + r, sl] = x
                yb_[yoff + r, sl] = d * x
          else:
            @plsc.parallel_loop(0, SH, unroll=2)
            def _(r):
              d = dinv_b[drow + r, :]
              for gq in range(HALF // L):
                sl = pl.ds(gq * L, L)
                ab[aoff + r, sl] = d * ab[aoff + r, sl]

          pltpu.async_copy(ab.at[pl.ds(aoff, SH), :],
                           out_hbm.at[pl.ds(r0, SH), pl.ds(col0, HALF)],
                           so[t])
          if k < KHOP:
            pltpu.async_copy(yb_.at[pl.ds(yoff, SH), :],
                             y_hbm.at[pl.ds(cN + r0, SH), :], sy[t])
          for z in range(SH // ZR):
            pltpu.async_copy(zb, acc_sp.at[pl.ds(r0 + z * ZR, ZR), :], zsem)
      return carry
    lax.fori_loop(0, (2 * MAXC + 2) // 3, pp_triple, 0)

    for t in range(3):
      ab, aoff = ACCH[t]
      yb_, yoff = YH[t]
      pltpu.make_async_copy(ab.at[pl.ds(aoff, SH), :],
                            out_hbm.at[pl.ds(0, SH), pl.ds(c * HALF, HALF)],
                            so[t]).wait()
      if k < KHOP:
        pltpu.make_async_copy(yb_.at[pl.ds(yoff, SH), :],
                              y_hbm.at[pl.ds(cN, SH), :], sy[t]).wait()

    def zdrain(q, carry):
      pltpu.make_async_copy(zb, acc_sp.at[pl.ds(0, ZR), :], zsem).wait()
      return carry
    lax.fori_loop(0, 2 * ns2, zdrain, 0)

    plsc.subcore_barrier()


@jax.jit
def _lgcn(feature, src, dst):
  mesh = plsc.VectorSubcoreMesh(core_axis_name="c", subcore_axis_name="s")
  out, _ = pl.kernel(
      _body,
      out_type=(
          jax.ShapeDtypeStruct((N, (KHOP + 1) * FD), jnp.float32),
          jax.ShapeDtypeStruct((NC * N, HALF), jnp.float32),
      ),
      mesh=mesh,
      compiler_params=pltpu.CompilerParams(use_tc_tiling_on_sc=False),
      scratch_types=[
          pltpu.VMEM_SHARED((N, HALF), jnp.float32),
          pltpu.VMEM((CH * 8, L), jnp.float32),
          pltpu.VMEM((EC, HALF), jnp.float32),
          pltpu.VMEM((EC, HALF), jnp.float32),
          pltpu.VMEM((G * EC,), jnp.int32),
          pltpu.VMEM((G * EC,), jnp.int32),
          pltpu.VMEM((G * EC,), jnp.int32),
          pltpu.VMEM((ZR, HALF), jnp.float32),
          pltpu.VMEM((CH, HALF), jnp.float32),
          pltpu.SemaphoreType.DMA,
          pltpu.SemaphoreType.DMA,
          pltpu.SemaphoreType.DMA,
          pltpu.SemaphoreType.DMA,
          pltpu.SemaphoreType.DMA,
          pltpu.SemaphoreType.DMA,
          pltpu.SemaphoreType.DMA,
          pltpu.SemaphoreType.DMA,
          pltpu.SemaphoreType.DMA,
          pltpu.SemaphoreType.DMA,
          pltpu.SemaphoreType.DMA,
          pltpu.SemaphoreType.DMA,
          pltpu.SemaphoreType.DMA,
          pltpu.SemaphoreType.DMA,
          pltpu.SemaphoreType.DMA,
          pltpu.SemaphoreType.DMA,
      ],
  )(feature, src, dst)
  return out


def kernel(feature, edge_index):
  ei = edge_index.astype(jnp.int32)
  return _lgcn(feature, ei[0], ei[1])


# speedup vs baseline: 15.0728x; 1.0530x over previous
import jax
import jax.numpy as jnp
from jax import lax
from jax.experimental import pallas as pl
from jax.experimental.pallas import tpu as pltpu
from jax.experimental.pallas import tpu_sc as plsc

N = 10000
FD = 256
HALF = 128
E = 160000
KHOP = 8
NS = 16
NC = 2
L = 16
CH = 80
NCH = N // CH
MAXC = 8
ZR = 20
SH = 40
EPT = E // NS
EC = 80
NEC = EPT // EC
G = 25
NG = NEC // G


def _body(f_hbm, src_hbm, dst_hbm, out_hbm, y_hbm,
          acc_sp,
          dinv_b, rows0, rows1, sbuf, dbuf, gbuf,
          zb, accb, sg0, sg1, sg2, ss0, ss1, ss2,
          sr0, sr1, sr2, so0, so1, so2, sy0, sy1, sy2, zsem):
  c = lax.axis_index("c")
  s = lax.axis_index("s")
  cN = c * N
  e_base = s * EPT
  GE = G * EC

  rows = (rows0, rows1)
  sg = (sg0, sg1)
  ss = (ss0, ss1)
  ROWS3 = (rows0, rows1, accb)
  sg3 = (sg0, sg1, sg2)
  ss3 = (ss0, ss1, ss2)
  sr = (sr0, sr1, sr2)
  so = (so0, so1, so2)
  sy = (sy0, sy1, sy2)
  ACCH = ((accb, 0), (accb, SH), (rows1, 0))
  YH = ((rows0, 0), (rows0, SH), (rows1, SH))

  fzero = jnp.zeros((L,), jnp.float32)
  fone = jnp.ones((L,), jnp.float32)
  iota = lax.iota(jnp.int32, L)

  def fill_zb(r, carry):
    for g in range(HALF // L):
      zb[r, pl.ds(g * L, L)] = fzero
    return carry
  lax.fori_loop(0, ZR, fill_zb, 0)

  def fill_ones(r, carry):
    for g in range(HALF // L):
      rows0[r, pl.ds(g * L, L)] = fone
    return carry
  lax.fori_loop(0, EC, fill_ones, 0)

  def zero_acc_chunk(r0):
    def zbody(z, carry):
      pltpu.sync_copy(zb, acc_sp.at[pl.ds(r0 + z * ZR, ZR), :])
      return carry
    lax.fori_loop(0, CH // ZR, zbody, 0)

  nchunks = (NCH - 1 - s) // NS + 1

  def zinit_body(i, carry):
    zero_acc_chunk((s + NS * i) * CH)
    return carry
  lax.fori_loop(0, nchunks, zinit_body, 0)

  plsc.subcore_barrier()

  def dslice(j):
    return dbuf.at[pl.ds(j * EC, EC)]

  def deg_drain3():
    for jj in (G - 3, G - 2, G - 1):
      pltpu.make_async_copy(rows0, acc_sp.at[dslice(jj)],
                            ss3[jj % 3]).wait()

  def deg_group(g, carry):
    @pl.when(g > 0)
    def _():
      deg_drain3()
    pltpu.sync_copy(dst_hbm.at[pl.ds(e_base + g * GE, GE)], dbuf)
    for j in range(G):
      t = j % 3
      if j >= 3:
        pltpu.make_async_copy(rows0, acc_sp.at[dslice(j - 3)],
                              ss3[t]).wait()
      pltpu.async_copy(rows0, acc_sp.at[dslice(j)], ss3[t], add=True)
    return carry
  lax.fori_loop(0, NG, deg_group, 0)
  deg_drain3()

  plsc.subcore_barrier()

  def dinv_chunk(i, carry):
    cid = s + NS * i
    pltpu.sync_copy(acc_sp.at[pl.ds(cid * CH, CH), :], accb)

    def dinv_body(r, rcarry):
      v = accb[r, pl.ds(0, L)] + 1.0
      g = jnp.full((L,), 1.0, jnp.float32)
      for j in range(1, 19):
        g = jnp.where(v >= float(1 << j),
                      jnp.full((L,), 2.0 ** (-0.5 * j), jnp.float32), g)
      for _ in range(5):
        g = g * (1.5 - 0.5 * v * g * g)
      dinv_b[i * CH + r, :] = g
      return rcarry
    lax.fori_loop(0, CH, dinv_body, 0)

    return carry
  lax.fori_loop(0, nchunks, dinv_chunk, 0)

  def p0_chunk(i, carry):
    r0 = (s + NS * i) * CH
    pltpu.sync_copy(f_hbm.at[pl.ds(r0, CH), pl.ds(c * HALF, HALF)], accb)

    @plsc.parallel_loop(0, CH, unroll=2)
    def _(r):
      d = dinv_b[i * CH + r, :]
      for g in range(HALF // L):
        sl = pl.ds(g * L, L)
        rows0[r, sl] = d * accb[r, sl]

    pltpu.sync_copy(accb, out_hbm.at[pl.ds(r0, CH), pl.ds(c * HALF, HALF)])
    pltpu.sync_copy(rows0, y_hbm.at[pl.ds(cN + r0, CH), :])
    pltpu.sync_copy(rows0, acc_sp.at[pl.ds(r0, CH), :])
    return carry
  lax.fori_loop(0, nchunks, p0_chunk, 0)

  plsc.subcore_barrier()

  for k in range(1, KHOP + 1):
    def gslice(j):
      return gbuf.at[pl.ds(j * EC, EC)]

    def drain3(tag):
      for jj in (G - 3, G - 2, G - 1):
        t = jj % 3
        pltpu.make_async_copy(ROWS3[t], acc_sp.at[dslice(jj)],
                              ss3[t]).wait()

    def edge_group(g, carry):
      @pl.when(g > 0)
      def _():
        drain3(0)
      e0 = e_base + g * GE
      pltpu.sync_copy(src_hbm.at[pl.ds(e0, GE)], sbuf)
      pltpu.sync_copy(dst_hbm.at[pl.ds(e0, GE)], dbuf)

      @plsc.parallel_loop(0, GE // L, unroll=4)
      def _(q):
        sl = pl.ds(q * L, L)
        gbuf[sl] = sbuf[sl] + cN

      pltpu.async_copy(y_hbm.at[gslice(0)], ROWS3[0], sg3[0])
      for j in range(1, G):
        t = j % 3
        tp = (j - 1) % 3
        if j >= 3:
          pltpu.make_async_copy(ROWS3[t], acc_sp.at[dslice(j - 3)],
                                ss3[t]).wait()
        pltpu.async_copy(y_hbm.at[gslice(j)], ROWS3[t], sg3[t])
        pltpu.make_async_copy(y_hbm.at[gslice(j - 1)], ROWS3[tp],
                              sg3[tp]).wait()
        pltpu.async_copy(ROWS3[tp], acc_sp.at[dslice(j - 1)], ss3[tp],
                         add=True)
      tl = (G - 1) % 3
      pltpu.make_async_copy(y_hbm.at[gslice(G - 1)], ROWS3[tl],
                            sg3[tl]).wait()
      pltpu.async_copy(ROWS3[tl], acc_sp.at[dslice(G - 1)], ss3[tl],
                       add=True)
      return carry
    lax.fori_loop(0, NG, edge_group, 0)
    drain3(1)

    plsc.subcore_barrier()

    col0 = k * FD + c * HALF
    ns2 = 2 * nchunks

    def stage_r0(m):
      return (s + NS * (m // 2)) * CH + (m % 2) * SH

    def issue_read(m, t):
      ab, off = ACCH[t]
      pltpu.async_copy(acc_sp.at[pl.ds(stage_r0(m), SH), :],
                       ab.at[pl.ds(off, SH), :], sr[t])

    issue_read(0, 0)

    def pp_triple(h, carry):
      for t in range(3):
        m = 3 * h + t

        @pl.when(m < ns2)
        def _():
          ab, aoff = ACCH[t]
          yb_, yoff = YH[t]
          t1 = (t + 1) % 3
          nab, naoff = ACCH[t1]
          r0 = stage_r0(m)

          @pl.when(m + 1 < ns2)
          def _():
            @pl.when(m + 1 >= 3)
            def _():
              pltpu.make_async_copy(
                  nab.at[pl.ds(naoff, SH), :],
                  out_hbm.at[pl.ds(0, SH), pl.ds(c * HALF, HALF)],
                  so[t1]).wait()
            issue_read(m + 1, t1)

          pltpu.make_async_copy(acc_sp.at[pl.ds(0, SH), :],
                                ab.at[pl.ds(aoff, SH), :], sr[t]).wait()

          if k < KHOP:
            @pl.when(m >= 3)
            def _():
              pltpu.make_async_copy(yb_.at[pl.ds(yoff, SH), :],
                                    y_hbm.at[pl.ds(cN, SH), :],
                                    sy[t]).wait()

          drow = (m // 2) * CH + (m % 2) * SH

          if k < KHOP:
            @plsc.parallel_loop(0, SH, unroll=2)
            def _(r):
              d = dinv_b[drow + r, :]
              for gq in range(HALF // L):
                sl = pl.ds(gq * L, L)
                x = d * ab[aoff + r, sl]
                ab[aoff + r, sl] = x
                yb_[yoff + r, sl] = d * x
          else:
            @plsc.parallel_loop(0, SH, unroll=2)
            def _(r):
              d = dinv_b[drow + r, :]
              for gq in range(HALF // L):
                sl = pl.ds(gq * L, L)
                ab[aoff + r, sl] = d * ab[aoff + r, sl]

          pltpu.async_copy(ab.at[pl.ds(aoff, SH), :],
                           out_hbm.at[pl.ds(r0, SH), pl.ds(col0, HALF)],
                           so[t])
          if k < KHOP:
            pltpu.async_copy(yb_.at[pl.ds(yoff, SH), :],
                             y_hbm.at[pl.ds(cN + r0, SH), :], sy[t])
            pltpu.sync_copy(yb_.at[pl.ds(yoff, SH), :],
                            acc_sp.at[pl.ds(r0, SH), :])
      return carry
    lax.fori_loop(0, (2 * MAXC + 2) // 3, pp_triple, 0)

    for t in range(3):
      ab, aoff = ACCH[t]
      yb_, yoff = YH[t]
      pltpu.make_async_copy(ab.at[pl.ds(aoff, SH), :],
                            out_hbm.at[pl.ds(0, SH), pl.ds(c * HALF, HALF)],
                            so[t]).wait()
      if k < KHOP:
        pltpu.make_async_copy(yb_.at[pl.ds(yoff, SH), :],
                              y_hbm.at[pl.ds(cN, SH), :], sy[t]).wait()

    plsc.subcore_barrier()


@jax.jit
def _lgcn(feature, src, dst):
  mesh = plsc.VectorSubcoreMesh(core_axis_name="c", subcore_axis_name="s")
  out, _ = pl.kernel(
      _body,
      out_type=(
          jax.ShapeDtypeStruct((N, (KHOP + 1) * FD), jnp.float32),
          jax.ShapeDtypeStruct((NC * N, HALF), jnp.float32),
      ),
      mesh=mesh,
      compiler_params=pltpu.CompilerParams(use_tc_tiling_on_sc=False),
      scratch_types=[
          pltpu.VMEM_SHARED((N, HALF), jnp.float32),
          pltpu.VMEM((CH * 8, L), jnp.float32),
          pltpu.VMEM((EC, HALF), jnp.float32),
          pltpu.VMEM((EC, HALF), jnp.float32),
          pltpu.VMEM((G * EC,), jnp.int32),
          pltpu.VMEM((G * EC,), jnp.int32),
          pltpu.VMEM((G * EC,), jnp.int32),
          pltpu.VMEM((ZR, HALF), jnp.float32),
          pltpu.VMEM((CH, HALF), jnp.float32),
          pltpu.SemaphoreType.DMA,
          pltpu.SemaphoreType.DMA,
          pltpu.SemaphoreType.DMA,
          pltpu.SemaphoreType.DMA,
          pltpu.SemaphoreType.DMA,
          pltpu.SemaphoreType.DMA,
          pltpu.SemaphoreType.DMA,
          pltpu.SemaphoreType.DMA,
          pltpu.SemaphoreType.DMA,
          pltpu.SemaphoreType.DMA,
          pltpu.SemaphoreType.DMA,
          pltpu.SemaphoreType.DMA,
          pltpu.SemaphoreType.DMA,
          pltpu.SemaphoreType.DMA,
          pltpu.SemaphoreType.DMA,
          pltpu.SemaphoreType.DMA,
      ],
  )(feature, src, dst)
  return out


def kernel(feature, edge_index):
  ei = edge_index.astype(jnp.int32)
  return _lgcn(feature, ei[0], ei[1])


# speedup vs baseline: 15.2726x; 1.0133x over previous
import jax
import jax.numpy as jnp
from jax import lax
from jax.experimental import pallas as pl
from jax.experimental.pallas import tpu as pltpu
from jax.experimental.pallas import tpu_sc as plsc

N = 10000
FD = 256
HALF = 128
E = 160000
KHOP = 8
NS = 16
NC = 2
L = 16
CH = 80
NCH = N // CH
MAXC = 8
ZR = 20
SH = 40
EPT = E // NS
EC = 80
NEC = EPT // EC
G = 25
NG = NEC // G


def _body(f_hbm, src_hbm, dst_hbm, out_hbm, y_hbm,
          acc_sp,
          dinv_b, rows0, rows1, sbuf, dbuf, gbuf,
          zb, accb, sg0, sg1, sg2, ss0, ss1, ss2,
          sr0, sr1, sr2, so0, so1, so2, sy0, sy1, sy2, zsem):
  c = lax.axis_index("c")
  s = lax.axis_index("s")
  cN = c * N
  e_base = s * EPT
  GE = G * EC

  rows = (rows0, rows1)
  sg = (sg0, sg1)
  ss = (ss0, ss1)
  ROWS3 = (rows0, rows1, accb)
  sg3 = (sg0, sg1, sg2)
  ss3 = (ss0, ss1, ss2)
  sr = (sr0, sr1, sr2)
  so = (so0, so1, so2)
  sy = (sy0, sy1, sy2)
  ACCH = ((accb, 0), (accb, SH), (rows1, 0))
  YH = ((rows0, 0), (rows0, SH), (rows1, SH))

  fzero = jnp.zeros((L,), jnp.float32)
  fone = jnp.ones((L,), jnp.float32)
  iota = lax.iota(jnp.int32, L)

  def fill_zb(r, carry):
    for g in range(HALF // L):
      zb[r, pl.ds(g * L, L)] = fzero
    return carry
  lax.fori_loop(0, ZR, fill_zb, 0)

  def fill_ones(r, carry):
    for g in range(HALF // L):
      rows0[r, pl.ds(g * L, L)] = fone
    return carry
  lax.fori_loop(0, EC, fill_ones, 0)

  def zero_acc_chunk(r0):
    def zbody(z, carry):
      pltpu.sync_copy(zb, acc_sp.at[pl.ds(r0 + z * ZR, ZR), :])
      return carry
    lax.fori_loop(0, CH // ZR, zbody, 0)

  nchunks = (NCH - 1 - s) // NS + 1

  def zinit_body(i, carry):
    zero_acc_chunk((s + NS * i) * CH)
    return carry
  lax.fori_loop(0, nchunks, zinit_body, 0)

  plsc.subcore_barrier()

  def dslice(j):
    return dbuf.at[pl.ds(j * EC, EC)]

  def deg_drain3():
    for jj in (G - 3, G - 2, G - 1):
      pltpu.make_async_copy(rows0, acc_sp.at[dslice(jj)],
                            ss3[jj % 3]).wait()

  def deg_group(g, carry):
    @pl.when(g > 0)
    def _():
      deg_drain3()
    pltpu.sync_copy(dst_hbm.at[pl.ds(e_base + g * GE, GE)], dbuf)
    for j in range(G):
      t = j % 3
      if j >= 3:
        pltpu.make_async_copy(rows0, acc_sp.at[dslice(j - 3)],
                              ss3[t]).wait()
      pltpu.async_copy(rows0, acc_sp.at[dslice(j)], ss3[t], add=True)
    return carry
  lax.fori_loop(0, NG, deg_group, 0)
  deg_drain3()

  plsc.subcore_barrier()

  def dinv_chunk(i, carry):
    cid = s + NS * i
    pltpu.sync_copy(acc_sp.at[pl.ds(cid * CH, CH), :], accb)

    def dinv_body(r, rcarry):
      v = accb[r, pl.ds(0, L)] + 1.0
      g = jnp.full((L,), 1.0, jnp.float32)
      for j in range(1, 19):
        g = jnp.where(v >= float(1 << j),
                      jnp.full((L,), 2.0 ** (-0.5 * j), jnp.float32), g)
      for _ in range(5):
        g = g * (1.5 - 0.5 * v * g * g)
      dinv_b[i * CH + r, :] = g
      return rcarry
    lax.fori_loop(0, CH, dinv_body, 0)

    return carry
  lax.fori_loop(0, nchunks, dinv_chunk, 0)

  def p0_chunk(i, carry):
    r0 = (s + NS * i) * CH
    pltpu.sync_copy(f_hbm.at[pl.ds(r0, CH), pl.ds(c * HALF, HALF)], accb)

    @plsc.parallel_loop(0, CH, unroll=2)
    def _(r):
      d = dinv_b[i * CH + r, :]
      for g in range(HALF // L):
        sl = pl.ds(g * L, L)
        rows0[r, sl] = d * accb[r, sl]

    pltpu.sync_copy(accb, out_hbm.at[pl.ds(r0, CH), pl.ds(c * HALF, HALF)])
    pltpu.sync_copy(rows0, y_hbm.at[pl.ds(cN + r0, CH), :])
    pltpu.sync_copy(rows0, acc_sp.at[pl.ds(r0, CH), :])
    return carry
  lax.fori_loop(0, nchunks, p0_chunk, 0)

  plsc.subcore_barrier()

  for k in range(1, KHOP + 1):
    def gslice(j):
      return gbuf.at[pl.ds(j * EC, EC)]

    def drain3(tag):
      for jj in (G - 3, G - 2, G - 1):
        t = jj % 3
        pltpu.make_async_copy(ROWS3[t], acc_sp.at[dslice(jj)],
                              ss3[t]).wait()

    def edge_group(g, carry):
      @pl.when(g > 0)
      def _():
        drain3(0)
      e0 = e_base + g * GE
      pltpu.sync_copy(src_hbm.at[pl.ds(e0, GE)], sbuf)
      pltpu.sync_copy(dst_hbm.at[pl.ds(e0, GE)], dbuf)

      @plsc.parallel_loop(0, GE // L, unroll=4)
      def _(q):
        sl = pl.ds(q * L, L)
        gbuf[sl] = sbuf[sl] + cN

      for j in range(G):
        t = j % 3
        if j >= 3:
          pltpu.make_async_copy(ROWS3[t], acc_sp.at[dslice(j - 3)],
                                ss3[t]).wait()
        pltpu.async_copy(y_hbm.at[gslice(j)], ROWS3[t], sg3[t])
        if j >= 2:
          tp = (j - 2) % 3
          pltpu.make_async_copy(y_hbm.at[gslice(j - 2)], ROWS3[tp],
                                sg3[tp]).wait()
          pltpu.async_copy(ROWS3[tp], acc_sp.at[dslice(j - 2)], ss3[tp],
                           add=True)
      for jj in (G - 2, G - 1):
        tp = jj % 3
        pltpu.make_async_copy(y_hbm.at[gslice(jj)], ROWS3[tp],
                              sg3[tp]).wait()
        pltpu.async_copy(ROWS3[tp], acc_sp.at[dslice(jj)], ss3[tp],
                         add=True)
      return carry
    lax.fori_loop(0, NG, edge_group, 0)
    drain3(1)

    plsc.subcore_barrier()

    col0 = k * FD + c * HALF
    ns2 = 2 * nchunks

    def stage_r0(m):
      return (s + NS * (m // 2)) * CH + (m % 2) * SH

    def issue_read(m, t):
      ab, off = ACCH[t]
      pltpu.async_copy(acc_sp.at[pl.ds(stage_r0(m), SH), :],
                       ab.at[pl.ds(off, SH), :], sr[t])

    issue_read(0, 0)

    def pp_triple(h, carry):
      for t in range(3):
        m = 3 * h + t

        @pl.when(m < ns2)
        def _():
          ab, aoff = ACCH[t]
          yb_, yoff = YH[t]
          t1 = (t + 1) % 3
          nab, naoff = ACCH[t1]
          r0 = stage_r0(m)

          @pl.when(m + 1 < ns2)
          def _():
            @pl.when(m + 1 >= 3)
            def _():
              pltpu.make_async_copy(
                  nab.at[pl.ds(naoff, SH), :],
                  out_hbm.at[pl.ds(0, SH), pl.ds(c * HALF, HALF)],
                  so[t1]).wait()
            issue_read(m + 1, t1)

          pltpu.make_async_copy(acc_sp.at[pl.ds(0, SH), :],
                                ab.at[pl.ds(aoff, SH), :], sr[t]).wait()

          if k < KHOP:
            @pl.when(m >= 3)
            def _():
              pltpu.make_async_copy(yb_.at[pl.ds(yoff, SH), :],
                                    y_hbm.at[pl.ds(cN, SH), :],
                                    sy[t]).wait()

          drow = (m // 2) * CH + (m % 2) * SH

          if k < KHOP:
            @plsc.parallel_loop(0, SH, unroll=2)
            def _(r):
              d = dinv_b[drow + r, :]
              for gq in range(HALF // L):
                sl = pl.ds(gq * L, L)
                x = d * ab[aoff + r, sl]
                ab[aoff + r, sl] = x
                yb_[yoff + r, sl] = d * x
          else:
            @plsc.parallel_loop(0, SH, unroll=2)
            def _(r):
              d = dinv_b[drow + r, :]
              for gq in range(HALF // L):
                sl = pl.ds(gq * L, L)
                ab[aoff + r, sl] = d * ab[aoff + r, sl]

          pltpu.async_copy(ab.at[pl.ds(aoff, SH), :],
                           out_hbm.at[pl.ds(r0, SH), pl.ds(col0, HALF)],
                           so[t])
          if k < KHOP:
            pltpu.async_copy(yb_.at[pl.ds(yoff, SH), :],
                             y_hbm.at[pl.ds(cN + r0, SH), :], sy[t])
            pltpu.sync_copy(yb_.at[pl.ds(yoff, SH), :],
                            acc_sp.at[pl.ds(r0, SH), :])
      return carry
    lax.fori_loop(0, (2 * MAXC + 2) // 3, pp_triple, 0)

    for t in range(3):
      ab, aoff = ACCH[t]
      yb_, yoff = YH[t]
      pltpu.make_async_copy(ab.at[pl.ds(aoff, SH), :],
                            out_hbm.at[pl.ds(0, SH), pl.ds(c * HALF, HALF)],
                            so[t]).wait()
      if k < KHOP:
        pltpu.make_async_copy(yb_.at[pl.ds(yoff, SH), :],
                              y_hbm.at[pl.ds(cN, SH), :], sy[t]).wait()

    plsc.subcore_barrier()


@jax.jit
def _lgcn(feature, src, dst):
  mesh = plsc.VectorSubcoreMesh(core_axis_name="c", subcore_axis_name="s")
  out, _ = pl.kernel(
      _body,
      out_type=(
          jax.ShapeDtypeStruct((N, (KHOP + 1) * FD), jnp.float32),
          jax.ShapeDtypeStruct((NC * N, HALF), jnp.float32),
      ),
      mesh=mesh,
      compiler_params=pltpu.CompilerParams(use_tc_tiling_on_sc=False),
      scratch_types=[
          pltpu.VMEM_SHARED((N, HALF), jnp.float32),
          pltpu.VMEM((CH * 8, L), jnp.float32),
          pltpu.VMEM((EC, HALF), jnp.float32),
          pltpu.VMEM((EC, HALF), jnp.float32),
          pltpu.VMEM((G * EC,), jnp.int32),
          pltpu.VMEM((G * EC,), jnp.int32),
          pltpu.VMEM((G * EC,), jnp.int32),
          pltpu.VMEM((ZR, HALF), jnp.float32),
          pltpu.VMEM((CH, HALF), jnp.float32),
          pltpu.SemaphoreType.DMA,
          pltpu.SemaphoreType.DMA,
          pltpu.SemaphoreType.DMA,
          pltpu.SemaphoreType.DMA,
          pltpu.SemaphoreType.DMA,
          pltpu.SemaphoreType.DMA,
          pltpu.SemaphoreType.DMA,
          pltpu.SemaphoreType.DMA,
          pltpu.SemaphoreType.DMA,
          pltpu.SemaphoreType.DMA,
          pltpu.SemaphoreType.DMA,
          pltpu.SemaphoreType.DMA,
          pltpu.SemaphoreType.DMA,
          pltpu.SemaphoreType.DMA,
          pltpu.SemaphoreType.DMA,
          pltpu.SemaphoreType.DMA,
      ],
  )(feature, src, dst)
  return out


def kernel(feature, edge_index):
  ei = edge_index.astype(jnp.int32)
  return _lgcn(feature, ei[0], ei[1])
